# initial kernel scaffold (unmeasured)
import jax
import jax.numpy as jnp
from jax import lax
from jax.experimental import pallas as pl
from jax.experimental.pallas import tpu as pltpu


def kernel(
    x,
):
    def body(*refs):
        pass

    out_shape = jax.ShapeDtypeStruct(..., jnp.float32)
    return pl.pallas_call(body, out_shape=out_shape)(...)


# baseline (device time: 31023 ns/iter reference)
import jax
import jax.numpy as jnp
from jax import lax
from jax.experimental import pallas as pl
from jax.experimental.pallas import tpu as pltpu


def kernel(x):
    m, n = x.shape

    def body(x_ref, out_ref, recv_buf, send_sems, recv_sems):
        my_x = lax.axis_index("x")
        my_y = lax.axis_index("y")
        x_nbr = (1 - my_x, my_y)
        y_nbr = (my_x, 1 - my_y)

        barrier_sem = pltpu.get_barrier_semaphore()
        for nbr in (x_nbr, y_nbr):
            pl.semaphore_signal(
                barrier_sem, inc=1,
                device_id=nbr, device_id_type=pl.DeviceIdType.MESH,
            )
        pl.semaphore_wait(barrier_sem, 2)

        out_ref[...] = x_ref[...]

        rdma0 = pltpu.make_async_remote_copy(
            src_ref=out_ref,
            dst_ref=recv_buf.at[0],
            send_sem=send_sems.at[0],
            recv_sem=recv_sems.at[0],
            device_id=x_nbr,
            device_id_type=pl.DeviceIdType.MESH,
        )
        rdma0.start()
        rdma0.wait()
        out_ref[...] += recv_buf[0]

        rdma1 = pltpu.make_async_remote_copy(
            src_ref=out_ref,
            dst_ref=recv_buf.at[1],
            send_sem=send_sems.at[1],
            recv_sem=recv_sems.at[1],
            device_id=y_nbr,
            device_id_type=pl.DeviceIdType.MESH,
        )
        rdma1.start()
        rdma1.wait()
        out_ref[...] += recv_buf[1]

    return pl.pallas_call(
        body,
        out_shape=jax.ShapeDtypeStruct((m, n), jnp.float32),
        in_specs=[pl.BlockSpec(memory_space=pltpu.VMEM)],
        out_specs=pl.BlockSpec(memory_space=pltpu.VMEM),
        scratch_shapes=[
            pltpu.VMEM((2, m, n), jnp.float32),
            pltpu.SemaphoreType.DMA((2,)),
            pltpu.SemaphoreType.DMA((2,)),
        ],
        compiler_params=pltpu.CompilerParams(collective_id=0),
    )(x)


# device time: 19866 ns/iter; 1.5616x vs baseline; 1.5616x over previous
import jax
import jax.numpy as jnp
from jax import lax
from jax.experimental import pallas as pl
from jax.experimental.pallas import tpu as pltpu


def kernel(x):
    m, n = x.shape
    h = m // 2

    def body(x_ref, out_ref, recv_buf, send_sems, recv_sems):
        my_x = lax.axis_index("x")
        my_y = lax.axis_index("y")
        x_nbr = (1 - my_x, my_y)
        y_nbr = (my_x, 1 - my_y)

        barrier_sem = pltpu.get_barrier_semaphore()
        for nbr in (x_nbr, y_nbr):
            pl.semaphore_signal(
                barrier_sem, inc=1,
                device_id=nbr, device_id_type=pl.DeviceIdType.MESH,
            )
        pl.semaphore_wait(barrier_sem, 2)

        out_ref[...] = x_ref[...]
        half = {"A": out_ref.at[pl.ds(0, h), :], "B": out_ref.at[pl.ds(h, h), :]}

        schedule = [
            (0, [("A", x_nbr), ("B", y_nbr)]),
            (1, [("A", y_nbr), ("B", x_nbr)]),
        ]
        for phase, xfers in schedule:
            rdmas = []
            for slot, (name, nbr) in enumerate(xfers):
                sem = 2 * phase + slot
                rdma = pltpu.make_async_remote_copy(
                    src_ref=half[name],
                    dst_ref=recv_buf.at[phase, slot],
                    send_sem=send_sems.at[sem],
                    recv_sem=recv_sems.at[sem],
                    device_id=nbr,
                    device_id_type=pl.DeviceIdType.MESH,
                )
                rdma.start()
                rdmas.append(rdma)
            for slot, (name, _) in enumerate(xfers):
                rdmas[slot].wait()
                half[name][...] += recv_buf[phase, slot]

    return pl.pallas_call(
        body,
        out_shape=jax.ShapeDtypeStruct((m, n), jnp.float32),
        in_specs=[pl.BlockSpec(memory_space=pltpu.VMEM)],
        out_specs=pl.BlockSpec(memory_space=pltpu.VMEM),
        scratch_shapes=[
            pltpu.VMEM((2, 2, h, n), jnp.float32),
            pltpu.SemaphoreType.DMA((4,)),
            pltpu.SemaphoreType.DMA((4,)),
        ],
        compiler_params=pltpu.CompilerParams(collective_id=0),
    )(x)


# device time: 18586 ns/iter; 1.6692x vs baseline; 1.0689x over previous
import jax
import jax.numpy as jnp
from jax import lax
from jax.experimental import pallas as pl
from jax.experimental.pallas import tpu as pltpu

N_CHUNKS = 4


def kernel(x):
    m, n = x.shape
    q = m // N_CHUNKS

    def body(x_ref, out_ref, recv_buf, send_sems, recv_sems):
        my_x = lax.axis_index("x")
        my_y = lax.axis_index("y")
        x_nbr = (1 - my_x, my_y)
        y_nbr = (my_x, 1 - my_y)

        barrier_sem = pltpu.get_barrier_semaphore()
        for nbr in (x_nbr, y_nbr):
            pl.semaphore_signal(
                barrier_sem, inc=1,
                device_id=nbr, device_id_type=pl.DeviceIdType.MESH,
            )
        pl.semaphore_wait(barrier_sem, 2)

        out_ref[...] = x_ref[...]

        def chunk(c):
            return out_ref.at[pl.ds(c * q, q), :]

        def mk_rdma(phase, c, nbr):
            sem = N_CHUNKS * phase + c
            return pltpu.make_async_remote_copy(
                src_ref=chunk(c),
                dst_ref=recv_buf.at[phase, c],
                send_sem=send_sems.at[sem],
                recv_sem=recv_sems.at[sem],
                device_id=nbr,
                device_id_type=pl.DeviceIdType.MESH,
            )

        nbr0 = [x_nbr, x_nbr, y_nbr, y_nbr]
        nbr1 = [y_nbr, y_nbr, x_nbr, x_nbr]
        order = [0, 2, 1, 3]

        p0 = {}
        for c in order:
            p0[c] = mk_rdma(0, c, nbr0[c])
            p0[c].start()
        p1 = {}
        for c in order:
            p0[c].wait()
            chunk(c)[...] += recv_buf[0, c]
            p1[c] = mk_rdma(1, c, nbr1[c])
            p1[c].start()
        for c in order:
            p1[c].wait()
            chunk(c)[...] += recv_buf[1, c]

    return pl.pallas_call(
        body,
        out_shape=jax.ShapeDtypeStruct((m, n), jnp.float32),
        in_specs=[pl.BlockSpec(memory_space=pltpu.VMEM)],
        out_specs=pl.BlockSpec(memory_space=pltpu.VMEM),
        scratch_shapes=[
            pltpu.VMEM((2, N_CHUNKS, q, n), jnp.float32),
            pltpu.SemaphoreType.DMA((2 * N_CHUNKS,)),
            pltpu.SemaphoreType.DMA((2 * N_CHUNKS,)),
        ],
        compiler_params=pltpu.CompilerParams(collective_id=0),
    )(x)
